# baseline (device time: 315501 ns/iter reference)
import jax
import jax.numpy as jnp
from jax import lax
from jax.experimental import pallas as pl
from jax.experimental.pallas import tpu as pltpu

N_DEV = 4
NSUB = 2


def kernel(x, w_mat):
    m, k_per = x.shape
    _, n = w_mat.shape
    mc = m // (2 * N_DEV)
    msc = mc // NSUB
    half = m // 2

    def top(c, j=None):
        if j is None:
            return pl.ds(c * mc, mc)
        return pl.ds(c * mc + j * msc, msc)

    def bot(c, j=None):
        if j is None:
            return pl.ds(half + c * mc, mc)
        return pl.ds(half + c * mc + j * msc, msc)

    def body(x_hbm, w_ref, out_ref, xcw, xccw, rcw, rccw, lsem1, lsem2,
             snd_cw, rcv_cw, snd_ccw, rcv_ccw, crd_cw, crd_ccw):
        my = lax.axis_index("i")
        left = lax.rem(my + N_DEV - 1, N_DEV)
        right = lax.rem(my + 1, N_DEV)

        def rs_send(dirn, s, j):
            c_cw = lax.rem(my + 2 * N_DEV - s, N_DEV)
            c_ccw = lax.rem(my + s, N_DEV)
            if dirn == 0:
                return pltpu.make_async_remote_copy(
                    src_ref=out_ref.at[top(c_cw, j), :],
                    dst_ref=rcw.at[s % 2, j],
                    send_sem=snd_cw.at[NSUB * s + j],
                    recv_sem=rcv_cw.at[NSUB * s + j],
                    device_id=(right,), device_id_type=pl.DeviceIdType.MESH,
                )
            return pltpu.make_async_remote_copy(
                src_ref=out_ref.at[bot(c_ccw, j), :],
                dst_ref=rccw.at[s % 2, j],
                send_sem=snd_ccw.at[NSUB * s + j],
                recv_sem=rcv_ccw.at[NSUB * s + j],
                device_id=(left,), device_id_type=pl.DeviceIdType.MESH,
            )

        def ag_send(dirn, t, j):
            k = NSUB * (N_DEV - 1) + NSUB * t + j
            g_cw = lax.rem(my + 2 * N_DEV + 1 - t, N_DEV)
            g_ccw = lax.rem(my + 2 * N_DEV - 1 + t, N_DEV)
            if dirn == 0:
                return pltpu.make_async_remote_copy(
                    src_ref=out_ref.at[top(g_cw, j), :],
                    dst_ref=out_ref.at[top(g_cw, j), :],
                    send_sem=snd_cw.at[k], recv_sem=rcv_cw.at[k],
                    device_id=(right,), device_id_type=pl.DeviceIdType.MESH,
                )
            return pltpu.make_async_remote_copy(
                src_ref=out_ref.at[bot(g_ccw, j), :],
                dst_ref=out_ref.at[bot(g_ccw, j), :],
                send_sem=snd_ccw.at[k], recv_sem=rcv_ccw.at[k],
                device_id=(left,), device_id_type=pl.DeviceIdType.MESH,
            )

        ld1 = pltpu.make_async_copy(
            x_hbm.at[pl.ds(my * mc, mc), :], xcw, lsem1)
        ld2 = pltpu.make_async_copy(
            x_hbm.at[pl.ds(half + my * mc, mc), :], xccw, lsem2)
        ld1.start()
        ld2.start()

        barrier_sem = pltpu.get_barrier_semaphore()
        for nbr in (left, right):
            pl.semaphore_signal(
                barrier_sem, inc=1,
                device_id=(nbr,), device_id_type=pl.DeviceIdType.MESH,
            )
        pl.semaphore_wait(barrier_sem, 2)

        ld1.wait()
        out_ref[top(my, 0), :] = jnp.dot(
            xcw[:msc], w_ref[...], preferred_element_type=jnp.float32)
        cur_aw = rs_send(0, 0, 0)
        cur_aw.start()
        ld2.wait()
        out_ref[bot(my, 0), :] = jnp.dot(
            xccw[:msc], w_ref[...], preferred_element_type=jnp.float32)
        cur_av = rs_send(1, 0, 0)
        cur_av.start()
        out_ref[top(my, 1), :] = jnp.dot(
            xcw[msc:], w_ref[...], preferred_element_type=jnp.float32)
        cur_bw = rs_send(0, 0, 1)
        cur_bw.start()
        out_ref[bot(my, 1), :] = jnp.dot(
            xccw[msc:], w_ref[...], preferred_element_type=jnp.float32)
        cur_bv = rs_send(1, 0, 1)
        cur_bv.start()

        for s in range(N_DEV - 1):
            r_cw = lax.rem(my + 2 * N_DEV - s - 1, N_DEV)
            r_ccw = lax.rem(my + s + 1, N_DEV)
            ld1 = pltpu.make_async_copy(
                x_hbm.at[pl.ds(r_cw * mc, mc), :], xcw, lsem1)
            ld2 = pltpu.make_async_copy(
                x_hbm.at[pl.ds(half + r_ccw * mc, mc), :], xccw, lsem2)
            ld1.start()
            ld2.start()
            ld1.wait()
            out_ref[top(r_cw), :] = jnp.dot(
                xcw[...], w_ref[...], preferred_element_type=jnp.float32)
            ld2.wait()
            out_ref[bot(r_ccw), :] = jnp.dot(
                xccw[...], w_ref[...], preferred_element_type=jnp.float32)
            cur_aw.wait()
            out_ref[top(r_cw, 0), :] += rcw[s % 2, 0]
            cur_av.wait()
            out_ref[bot(r_ccw, 0), :] += rccw[s % 2, 0]
            if s == 0:
                pl.semaphore_signal(
                    crd_cw.at[0], inc=1, device_id=(left,),
                    device_id_type=pl.DeviceIdType.MESH)
                pl.semaphore_signal(
                    crd_ccw.at[0], inc=1, device_id=(right,),
                    device_id_type=pl.DeviceIdType.MESH)
            if s < N_DEV - 2:
                if s + 1 == 2:
                    pl.semaphore_wait(crd_cw.at[0], 1)
                cur_aw = rs_send(0, s + 1, 0)
                cur_aw.start()
                if s + 1 == 2:
                    pl.semaphore_wait(crd_ccw.at[0], 1)
                cur_av = rs_send(1, s + 1, 0)
                cur_av.start()
                cur_bw.wait()
                out_ref[top(r_cw, 1), :] += rcw[s % 2, 1]
                cur_bv.wait()
                out_ref[bot(r_ccw, 1), :] += rccw[s % 2, 1]
                if s == 0:
                    pl.semaphore_signal(
                        crd_cw.at[1], inc=1, device_id=(left,),
                        device_id_type=pl.DeviceIdType.MESH)
                    pl.semaphore_signal(
                        crd_ccw.at[1], inc=1, device_id=(right,),
                        device_id_type=pl.DeviceIdType.MESH)
                if s + 1 == 2:
                    pl.semaphore_wait(crd_cw.at[1], 1)
                cur_bw = rs_send(0, s + 1, 1)
                cur_bw.start()
                if s + 1 == 2:
                    pl.semaphore_wait(crd_ccw.at[1], 1)
                cur_bv = rs_send(1, s + 1, 1)
                cur_bv.start()

        sends = []
        agA = (ag_send(0, 0, 0), ag_send(1, 0, 0))
        agA[0].start()
        agA[1].start()
        r_cw = lax.rem(my + N_DEV + 1, N_DEV)
        r_ccw = lax.rem(my + N_DEV - 1, N_DEV)
        cur_bw.wait()
        out_ref[top(r_cw, 1), :] += rcw[(N_DEV - 2) % 2, 1]
        cur_bv.wait()
        out_ref[bot(r_ccw, 1), :] += rccw[(N_DEV - 2) % 2, 1]
        agB = (ag_send(0, 0, 1), ag_send(1, 0, 1))
        agB[0].start()
        agB[1].start()
        sends += list(agA) + list(agB)

        for t in range(1, N_DEV - 1):
            agA[0].wait_recv()
            agA[1].wait_recv()
            agA = (ag_send(0, t, 0), ag_send(1, t, 0))
            agA[0].start()
            agA[1].start()
            agB[0].wait_recv()
            agB[1].wait_recv()
            agB = (ag_send(0, t, 1), ag_send(1, t, 1))
            agB[0].start()
            agB[1].start()
            sends += list(agA) + list(agB)
        agA[0].wait_recv()
        agA[1].wait_recv()
        agB[0].wait_recv()
        agB[1].wait_recv()
        for rd in sends:
            rd.wait_send()

    nsems = 2 * NSUB * (N_DEV - 1)
    return pl.pallas_call(
        body,
        out_shape=jax.ShapeDtypeStruct((m, n), jnp.float32),
        in_specs=[
            pl.BlockSpec(memory_space=pl.ANY),
            pl.BlockSpec(memory_space=pltpu.VMEM),
        ],
        out_specs=pl.BlockSpec(memory_space=pltpu.VMEM),
        scratch_shapes=[
            pltpu.VMEM((mc, k_per), jnp.float32),
            pltpu.VMEM((mc, k_per), jnp.float32),
            pltpu.VMEM((2, NSUB, msc, n), jnp.float32),
            pltpu.VMEM((2, NSUB, msc, n), jnp.float32),
            pltpu.SemaphoreType.DMA,
            pltpu.SemaphoreType.DMA,
            pltpu.SemaphoreType.DMA((nsems,)),
            pltpu.SemaphoreType.DMA((nsems,)),
            pltpu.SemaphoreType.DMA((nsems,)),
            pltpu.SemaphoreType.DMA((nsems,)),
            pltpu.SemaphoreType.REGULAR((NSUB,)),
            pltpu.SemaphoreType.REGULAR((NSUB,)),
        ],
        compiler_params=pltpu.CompilerParams(
            collective_id=0,
            vmem_limit_bytes=62 * 1024 * 1024,
        ),
    )(x, w_mat)


# device time: 304859 ns/iter; 1.0349x vs baseline; 1.0349x over previous
import jax
import jax.numpy as jnp
from jax import lax
from jax.experimental import pallas as pl
from jax.experimental.pallas import tpu as pltpu

N_DEV = 4
NSUB = 2


def kernel(x, w_mat):
    m, k_per = x.shape
    _, n = w_mat.shape
    mc = m // (2 * N_DEV)
    msc = mc // NSUB
    half = m // 2

    def top(c, j):
        return pl.ds(c * mc + j * msc, msc)

    def bot(c, j):
        return pl.ds(half + c * mc + j * msc, msc)

    def body(x_hbm, w_ref, out_hbm, xcw, xccw, scw, sccw, pcw, pccw,
             rcw, rccw, lsem1, lsem2, csems,
             snd_cw, rcv_cw, snd_ccw, rcv_ccw):
        my = lax.axis_index("i")
        left = lax.rem(my + N_DEV - 1, N_DEV)
        right = lax.rem(my + 1, N_DEV)

        def rs_send(dirn, s, j):
            if dirn == 0:
                return pltpu.make_async_remote_copy(
                    src_ref=scw.at[j], dst_ref=rcw.at[s, j],
                    send_sem=snd_cw.at[NSUB * s + j],
                    recv_sem=rcv_cw.at[NSUB * s + j],
                    device_id=(right,), device_id_type=pl.DeviceIdType.MESH,
                )
            return pltpu.make_async_remote_copy(
                src_ref=sccw.at[j], dst_ref=rccw.at[s, j],
                send_sem=snd_ccw.at[NSUB * s + j],
                recv_sem=rcv_ccw.at[NSUB * s + j],
                device_id=(left,), device_id_type=pl.DeviceIdType.MESH,
            )

        def ag_send(dirn, t, j):
            k = NSUB * (N_DEV - 1) + NSUB * t + j
            last = t == N_DEV - 2
            g_cw = lax.rem(my + 2 * N_DEV + 1 - t, N_DEV)
            g_ccw = lax.rem(my + 2 * N_DEV - 1 + t, N_DEV)
            if dirn == 0:
                return pltpu.make_async_remote_copy(
                    src_ref=(scw.at[j] if t == 0 else rcw.at[t - 1, j]),
                    dst_ref=(out_hbm.at[top(g_cw, j), :] if last
                             else rcw.at[t, j]),
                    send_sem=snd_cw.at[k], recv_sem=rcv_cw.at[k],
                    device_id=(right,), device_id_type=pl.DeviceIdType.MESH,
                )
            return pltpu.make_async_remote_copy(
                src_ref=(sccw.at[j] if t == 0 else rccw.at[t - 1, j]),
                dst_ref=(out_hbm.at[bot(g_ccw, j), :] if last
                         else rccw.at[t, j]),
                send_sem=snd_ccw.at[k], recv_sem=rcv_ccw.at[k],
                device_id=(left,), device_id_type=pl.DeviceIdType.MESH,
            )

        ld1 = pltpu.make_async_copy(
            x_hbm.at[pl.ds(my * mc, mc), :], xcw, lsem1)
        ld2 = pltpu.make_async_copy(
            x_hbm.at[pl.ds(half + my * mc, mc), :], xccw, lsem2)
        ld1.start()
        ld2.start()

        barrier_sem = pltpu.get_barrier_semaphore()
        for nbr in (left, right):
            pl.semaphore_signal(
                barrier_sem, inc=1,
                device_id=(nbr,), device_id_type=pl.DeviceIdType.MESH,
            )
        pl.semaphore_wait(barrier_sem, 2)

        ld1.wait()
        scw[0] = jnp.dot(
            xcw[:msc], w_ref[...], preferred_element_type=jnp.float32)
        cur_aw = rs_send(0, 0, 0)
        cur_aw.start()
        ld2.wait()
        sccw[0] = jnp.dot(
            xccw[:msc], w_ref[...], preferred_element_type=jnp.float32)
        cur_av = rs_send(1, 0, 0)
        cur_av.start()
        scw[1] = jnp.dot(
            xcw[msc:], w_ref[...], preferred_element_type=jnp.float32)
        cur_bw = rs_send(0, 0, 1)
        cur_bw.start()
        sccw[1] = jnp.dot(
            xccw[msc:], w_ref[...], preferred_element_type=jnp.float32)
        cur_bv = rs_send(1, 0, 1)
        cur_bv.start()

        for s in range(N_DEV - 1):
            r_cw = lax.rem(my + 2 * N_DEV - s - 1, N_DEV)
            r_ccw = lax.rem(my + s + 1, N_DEV)
            ld1 = pltpu.make_async_copy(
                x_hbm.at[pl.ds(r_cw * mc, mc), :], xcw, lsem1)
            ld2 = pltpu.make_async_copy(
                x_hbm.at[pl.ds(half + r_ccw * mc, mc), :], xccw, lsem2)
            ld1.start()
            ld2.start()
            ld1.wait()
            pcw[...] = jnp.dot(
                xcw[...], w_ref[...], preferred_element_type=jnp.float32)
            ld2.wait()
            pccw[...] = jnp.dot(
                xccw[...], w_ref[...], preferred_element_type=jnp.float32)
            cur_aw.wait()
            scw[0] = pcw[:msc] + rcw[s, 0]
            cur_av.wait()
            sccw[0] = pccw[:msc] + rccw[s, 0]
            if s < N_DEV - 2:
                cur_aw = rs_send(0, s + 1, 0)
                cur_av = rs_send(1, s + 1, 0)
                cur_aw.start()
                cur_av.start()
                cur_bw.wait()
                scw[1] = pcw[msc:] + rcw[s, 1]
                cur_bv.wait()
                sccw[1] = pccw[msc:] + rccw[s, 1]
                cur_bw = rs_send(0, s + 1, 1)
                cur_bv = rs_send(1, s + 1, 1)
                cur_bw.start()
                cur_bv.start()

        o_cw = lax.rem(my + 1, N_DEV)
        o_ccw = lax.rem(my + N_DEV - 1, N_DEV)
        sends = []
        agA = (ag_send(0, 0, 0), ag_send(1, 0, 0))
        agA[0].start()
        agA[1].start()
        cur_bw.wait()
        scw[1] = pcw[msc:] + rcw[N_DEV - 2, 1]
        cur_bv.wait()
        sccw[1] = pccw[msc:] + rccw[N_DEV - 2, 1]
        agB = (ag_send(0, 0, 1), ag_send(1, 0, 1))
        agB[0].start()
        agB[1].start()
        sends += list(agA) + list(agB)

        copies = []
        for j in range(NSUB):
            st1 = pltpu.make_async_copy(
                scw.at[j], out_hbm.at[top(o_cw, j), :],
                csems.at[4 * (N_DEV - 1) + 2 * j])
            st2 = pltpu.make_async_copy(
                sccw.at[j], out_hbm.at[bot(o_ccw, j), :],
                csems.at[4 * (N_DEV - 1) + 2 * j + 1])
            st1.start()
            st2.start()
            copies += [st1, st2]

        def ag_copy(t, j):
            g_cw = lax.rem(my + 2 * N_DEV - t, N_DEV)
            g_ccw = lax.rem(my + t, N_DEV)
            cp1 = pltpu.make_async_copy(
                rcw.at[t, j], out_hbm.at[top(g_cw, j), :],
                csems.at[4 * t + 2 * j])
            cp2 = pltpu.make_async_copy(
                rccw.at[t, j], out_hbm.at[bot(g_ccw, j), :],
                csems.at[4 * t + 2 * j + 1])
            cp1.start()
            cp2.start()
            copies.extend([cp1, cp2])

        for t in range(1, N_DEV - 1):
            agA[0].wait_recv()
            agA[1].wait_recv()
            agA = (ag_send(0, t, 0), ag_send(1, t, 0))
            agA[0].start()
            agA[1].start()
            ag_copy(t - 1, 0)
            agB[0].wait_recv()
            agB[1].wait_recv()
            agB = (ag_send(0, t, 1), ag_send(1, t, 1))
            agB[0].start()
            agB[1].start()
            ag_copy(t - 1, 1)
            sends += list(agA) + list(agB)
        agA[0].wait_recv()
        agA[1].wait_recv()
        agB[0].wait_recv()
        agB[1].wait_recv()
        for rd in sends:
            rd.wait_send()
        for st in copies:
            st.wait()

    nsems = 2 * NSUB * (N_DEV - 1)
    return pl.pallas_call(
        body,
        out_shape=jax.ShapeDtypeStruct((m, n), jnp.float32),
        in_specs=[
            pl.BlockSpec(memory_space=pl.ANY),
            pl.BlockSpec(memory_space=pltpu.VMEM),
        ],
        out_specs=pl.BlockSpec(memory_space=pl.ANY),
        scratch_shapes=[
            pltpu.VMEM((mc, k_per), jnp.float32),
            pltpu.VMEM((mc, k_per), jnp.float32),
            pltpu.VMEM((NSUB, msc, n), jnp.float32),
            pltpu.VMEM((NSUB, msc, n), jnp.float32),
            pltpu.VMEM((mc, n), jnp.float32),
            pltpu.VMEM((mc, n), jnp.float32),
            pltpu.VMEM((N_DEV - 1, NSUB, msc, n), jnp.float32),
            pltpu.VMEM((N_DEV - 1, NSUB, msc, n), jnp.float32),
            pltpu.SemaphoreType.DMA,
            pltpu.SemaphoreType.DMA,
            pltpu.SemaphoreType.DMA((4 * (N_DEV - 1) + 4,)),
            pltpu.SemaphoreType.DMA((nsems,)),
            pltpu.SemaphoreType.DMA((nsems,)),
            pltpu.SemaphoreType.DMA((nsems,)),
            pltpu.SemaphoreType.DMA((nsems,)),
        ],
        compiler_params=pltpu.CompilerParams(
            collective_id=0,
            vmem_limit_bytes=56 * 1024 * 1024,
        ),
    )(x, w_mat)


# device time: 304343 ns/iter; 1.0367x vs baseline; 1.0017x over previous
import jax
import jax.numpy as jnp
from jax import lax
from jax.experimental import pallas as pl
from jax.experimental.pallas import tpu as pltpu

N_DEV = 4
NSUB = 4


def kernel(x, w_mat):
    m, k_per = x.shape
    _, n = w_mat.shape
    mc = m // (2 * N_DEV)
    msc = mc // NSUB
    half = m // 2

    def top(c, j):
        return pl.ds(c * mc + j * msc, msc)

    def bot(c, j):
        return pl.ds(half + c * mc + j * msc, msc)

    def body(x_hbm, w_ref, out_hbm, xcw, xccw, scw, sccw, pcw, pccw,
             rcw, rccw, lsem1, lsem2, csems,
             snd_cw, rcv_cw, snd_ccw, rcv_ccw):
        my = lax.axis_index("i")
        left = lax.rem(my + N_DEV - 1, N_DEV)
        right = lax.rem(my + 1, N_DEV)

        def rs_send(dirn, s, j):
            if dirn == 0:
                return pltpu.make_async_remote_copy(
                    src_ref=scw.at[j], dst_ref=rcw.at[s, j],
                    send_sem=snd_cw.at[NSUB * s + j],
                    recv_sem=rcv_cw.at[NSUB * s + j],
                    device_id=(right,), device_id_type=pl.DeviceIdType.MESH,
                )
            return pltpu.make_async_remote_copy(
                src_ref=sccw.at[j], dst_ref=rccw.at[s, j],
                send_sem=snd_ccw.at[NSUB * s + j],
                recv_sem=rcv_ccw.at[NSUB * s + j],
                device_id=(left,), device_id_type=pl.DeviceIdType.MESH,
            )

        def ag_send(dirn, t, j):
            k = NSUB * (N_DEV - 1) + NSUB * t + j
            last = t == N_DEV - 2
            g_cw = lax.rem(my + 2 * N_DEV + 1 - t, N_DEV)
            g_ccw = lax.rem(my + 2 * N_DEV - 1 + t, N_DEV)
            if dirn == 0:
                return pltpu.make_async_remote_copy(
                    src_ref=(scw.at[j] if t == 0 else rcw.at[t - 1, j]),
                    dst_ref=(out_hbm.at[top(g_cw, j), :] if last
                             else rcw.at[t, j]),
                    send_sem=snd_cw.at[k], recv_sem=rcv_cw.at[k],
                    device_id=(right,), device_id_type=pl.DeviceIdType.MESH,
                )
            return pltpu.make_async_remote_copy(
                src_ref=(sccw.at[j] if t == 0 else rccw.at[t - 1, j]),
                dst_ref=(out_hbm.at[bot(g_ccw, j), :] if last
                         else rccw.at[t, j]),
                send_sem=snd_ccw.at[k], recv_sem=rcv_ccw.at[k],
                device_id=(left,), device_id_type=pl.DeviceIdType.MESH,
            )

        ld1 = pltpu.make_async_copy(
            x_hbm.at[pl.ds(my * mc, mc), :], xcw, lsem1)
        ld2 = pltpu.make_async_copy(
            x_hbm.at[pl.ds(half + my * mc, mc), :], xccw, lsem2)
        ld1.start()
        ld2.start()

        barrier_sem = pltpu.get_barrier_semaphore()
        for nbr in (left, right):
            pl.semaphore_signal(
                barrier_sem, inc=1,
                device_id=(nbr,), device_id_type=pl.DeviceIdType.MESH,
            )
        pl.semaphore_wait(barrier_sem, 2)

        curw = [None] * NSUB
        curv = [None] * NSUB
        ld1.wait()
        scw[0] = jnp.dot(
            xcw[:msc], w_ref[...], preferred_element_type=jnp.float32)
        curw[0] = rs_send(0, 0, 0)
        curw[0].start()
        ld2.wait()
        sccw[0] = jnp.dot(
            xccw[:msc], w_ref[...], preferred_element_type=jnp.float32)
        curv[0] = rs_send(1, 0, 0)
        curv[0].start()
        for j in range(1, NSUB):
            scw[j] = jnp.dot(
                xcw[j * msc:(j + 1) * msc], w_ref[...],
                preferred_element_type=jnp.float32)
            curw[j] = rs_send(0, 0, j)
            curw[j].start()
            sccw[j] = jnp.dot(
                xccw[j * msc:(j + 1) * msc], w_ref[...],
                preferred_element_type=jnp.float32)
            curv[j] = rs_send(1, 0, j)
            curv[j].start()

        for s in range(N_DEV - 1):
            r_cw = lax.rem(my + 2 * N_DEV - s - 1, N_DEV)
            r_ccw = lax.rem(my + s + 1, N_DEV)
            ld1 = pltpu.make_async_copy(
                x_hbm.at[pl.ds(r_cw * mc, mc), :], xcw, lsem1)
            ld2 = pltpu.make_async_copy(
                x_hbm.at[pl.ds(half + r_ccw * mc, mc), :], xccw, lsem2)
            ld1.start()
            ld2.start()
            ld1.wait()
            pcw[...] = jnp.dot(
                xcw[...], w_ref[...], preferred_element_type=jnp.float32)
            ld2.wait()
            pccw[...] = jnp.dot(
                xccw[...], w_ref[...], preferred_element_type=jnp.float32)
            last_step = s == N_DEV - 2
            for j in range(NSUB):
                if last_step and j > 0:
                    break
                curw[j].wait()
                scw[j] = pcw[j * msc:(j + 1) * msc] + rcw[s, j]
                curv[j].wait()
                sccw[j] = pccw[j * msc:(j + 1) * msc] + rccw[s, j]
                if not last_step:
                    curw[j] = rs_send(0, s + 1, j)
                    curw[j].start()
                    curv[j] = rs_send(1, s + 1, j)
                    curv[j].start()

        sends = []
        agA = [None] * NSUB
        agA[0] = (ag_send(0, 0, 0), ag_send(1, 0, 0))
        agA[0][0].start()
        agA[0][1].start()
        s_last = N_DEV - 2
        for j in range(1, NSUB):
            curw[j].wait()
            scw[j] = pcw[j * msc:(j + 1) * msc] + rcw[s_last, j]
            curv[j].wait()
            sccw[j] = pccw[j * msc:(j + 1) * msc] + rccw[s_last, j]
            agA[j] = (ag_send(0, 0, j), ag_send(1, 0, j))
            agA[j][0].start()
            agA[j][1].start()
        sends += [d for pair in agA for d in pair]

        o_cw = lax.rem(my + 1, N_DEV)
        o_ccw = lax.rem(my + N_DEV - 1, N_DEV)
        n_agcopy = 2 * NSUB * (N_DEV - 2)
        copies = []
        for j in range(NSUB):
            st1 = pltpu.make_async_copy(
                scw.at[j], out_hbm.at[top(o_cw, j), :],
                csems.at[n_agcopy + 2 * j])
            st2 = pltpu.make_async_copy(
                sccw.at[j], out_hbm.at[bot(o_ccw, j), :],
                csems.at[n_agcopy + 2 * j + 1])
            st1.start()
            st2.start()
            copies += [st1, st2]

        def ag_copy(t, j):
            g_cw = lax.rem(my + 2 * N_DEV - t, N_DEV)
            g_ccw = lax.rem(my + t, N_DEV)
            cp1 = pltpu.make_async_copy(
                rcw.at[t, j], out_hbm.at[top(g_cw, j), :],
                csems.at[2 * NSUB * t + 2 * j])
            cp2 = pltpu.make_async_copy(
                rccw.at[t, j], out_hbm.at[bot(g_ccw, j), :],
                csems.at[2 * NSUB * t + 2 * j + 1])
            cp1.start()
            cp2.start()
            copies.extend([cp1, cp2])

        for t in range(1, N_DEV - 1):
            for j in range(NSUB):
                agA[j][0].wait_recv()
                agA[j][1].wait_recv()
                agA[j] = (ag_send(0, t, j), ag_send(1, t, j))
                agA[j][0].start()
                agA[j][1].start()
                ag_copy(t - 1, j)
            sends += [d for pair in agA for d in pair]
        for j in range(NSUB):
            agA[j][0].wait_recv()
            agA[j][1].wait_recv()
        for rd in sends:
            rd.wait_send()
        for st in copies:
            st.wait()

    nsems = 2 * NSUB * (N_DEV - 1)
    ncsems = 2 * NSUB * (N_DEV - 2) + 2 * NSUB
    return pl.pallas_call(
        body,
        out_shape=jax.ShapeDtypeStruct((m, n), jnp.float32),
        in_specs=[
            pl.BlockSpec(memory_space=pl.ANY),
            pl.BlockSpec(memory_space=pltpu.VMEM),
        ],
        out_specs=pl.BlockSpec(memory_space=pl.ANY),
        scratch_shapes=[
            pltpu.VMEM((mc, k_per), jnp.float32),
            pltpu.VMEM((mc, k_per), jnp.float32),
            pltpu.VMEM((NSUB, msc, n), jnp.float32),
            pltpu.VMEM((NSUB, msc, n), jnp.float32),
            pltpu.VMEM((mc, n), jnp.float32),
            pltpu.VMEM((mc, n), jnp.float32),
            pltpu.VMEM((N_DEV - 1, NSUB, msc, n), jnp.float32),
            pltpu.VMEM((N_DEV - 1, NSUB, msc, n), jnp.float32),
            pltpu.SemaphoreType.DMA,
            pltpu.SemaphoreType.DMA,
            pltpu.SemaphoreType.DMA((ncsems,)),
            pltpu.SemaphoreType.DMA((nsems,)),
            pltpu.SemaphoreType.DMA((nsems,)),
            pltpu.SemaphoreType.DMA((nsems,)),
            pltpu.SemaphoreType.DMA((nsems,)),
        ],
        compiler_params=pltpu.CompilerParams(
            collective_id=0,
            vmem_limit_bytes=56 * 1024 * 1024,
        ),
    )(x, w_mat)
